# baseline (speedup 1.0000x reference)
import jax
import jax.numpy as jnp
from jax.experimental import pallas as pl
from jax.experimental.pallas import tpu as pltpu

_CHUNK_ROWS = 2048


def _body(i_ref, a_ref, x_hbm, o_hbm, xbuf, obuf, insem, outsem):
    n = x_hbm.shape[0]
    c = _CHUNK_ROWS
    nchunks = n // c

    def get_in(k, slot):
        return pltpu.make_async_copy(
            x_hbm.at[pl.ds(k * c, c), :], xbuf.at[slot], insem.at[slot]
        )

    def put_out(k, slot):
        return pltpu.make_async_copy(
            obuf.at[slot], o_hbm.at[pl.ds(k * c, c), :], outsem.at[slot]
        )

    get_in(0, 0).start()
    for k in range(nchunks):
        slot = k % 2
        if k + 1 < nchunks:
            get_in(k + 1, (k + 1) % 2).start()
        get_in(k, slot).wait()
        if k >= 2:
            put_out(k - 2, slot).wait()
        x = xbuf[slot]
        idx = jnp.broadcast_to(i_ref[0:1, :], x.shape)
        z = jnp.take_along_axis(x, idx, axis=1) - a_ref[0:1, :]
        obuf[slot] = jax.nn.sigmoid(z)
        put_out(k, slot).start()
    if nchunks >= 2:
        put_out(nchunks - 2, (nchunks - 2) % 2).wait()
    put_out(nchunks - 1, (nchunks - 1) % 2).wait()


@jax.jit
def kernel(x, a, a_index):
    n, d = x.shape
    idx_b = jnp.broadcast_to(a_index[None, :], (8, d))
    a_b = jnp.broadcast_to(a, (8, d))
    return pl.pallas_call(
        _body,
        in_specs=[
            pl.BlockSpec((8, d), lambda: (0, 0)),
            pl.BlockSpec((8, d), lambda: (0, 0)),
            pl.BlockSpec(memory_space=pl.ANY),
        ],
        out_specs=pl.BlockSpec(memory_space=pl.ANY),
        out_shape=jax.ShapeDtypeStruct((n, d), x.dtype),
        scratch_shapes=[
            pltpu.VMEM((2, _CHUNK_ROWS, d), x.dtype),
            pltpu.VMEM((2, _CHUNK_ROWS, d), x.dtype),
            pltpu.SemaphoreType.DMA((2,)),
            pltpu.SemaphoreType.DMA((2,)),
        ],
    )(idx_b, a_b, x)


# speedup vs baseline: 1.1924x; 1.1924x over previous
import jax
import jax.numpy as jnp
from jax.experimental import pallas as pl
from jax.experimental.pallas import tpu as pltpu

_CHUNK_ROWS = 2048
_NBUF = 4


def _body(i_ref, a_ref, x_hbm, o_hbm, xbuf, obuf, insem, outsem):
    n = x_hbm.shape[0]
    c = _CHUNK_ROWS
    nchunks = n // c

    def get_in(k, slot):
        return pltpu.make_async_copy(
            x_hbm.at[pl.ds(k * c, c), :], xbuf.at[slot], insem.at[slot]
        )

    def put_out(k, slot):
        return pltpu.make_async_copy(
            obuf.at[slot], o_hbm.at[pl.ds(k * c, c), :], outsem.at[slot]
        )

    nbuf = _NBUF
    for k in range(min(nbuf - 1, nchunks)):
        get_in(k, k % nbuf).start()
    for k in range(nchunks):
        slot = k % nbuf
        if k + nbuf - 1 < nchunks:
            get_in(k + nbuf - 1, (k + nbuf - 1) % nbuf).start()
        get_in(k, slot).wait()
        if k >= nbuf:
            put_out(k - nbuf, slot).wait()
        x = xbuf[slot]
        idx = jnp.broadcast_to(i_ref[0:1, :], x.shape)
        z = jnp.take_along_axis(x, idx, axis=1) - a_ref[0:1, :]
        obuf[slot] = jax.nn.sigmoid(z)
        put_out(k, slot).start()
    for k in range(max(0, nchunks - nbuf), nchunks):
        put_out(k, k % nbuf).wait()


@jax.jit
def kernel(x, a, a_index):
    n, d = x.shape
    idx_b = jnp.broadcast_to(a_index[None, :], (8, d))
    a_b = jnp.broadcast_to(a, (8, d))
    return pl.pallas_call(
        _body,
        in_specs=[
            pl.BlockSpec((8, d), lambda: (0, 0)),
            pl.BlockSpec((8, d), lambda: (0, 0)),
            pl.BlockSpec(memory_space=pl.ANY),
        ],
        out_specs=pl.BlockSpec(memory_space=pl.ANY),
        out_shape=jax.ShapeDtypeStruct((n, d), x.dtype),
        scratch_shapes=[
            pltpu.VMEM((_NBUF, _CHUNK_ROWS, d), x.dtype),
            pltpu.VMEM((_NBUF, _CHUNK_ROWS, d), x.dtype),
            pltpu.SemaphoreType.DMA((_NBUF,)),
            pltpu.SemaphoreType.DMA((_NBUF,)),
        ],
    )(idx_b, a_b, x)


# speedup vs baseline: 1.3703x; 1.1492x over previous
import jax
import jax.numpy as jnp
from jax.experimental import pallas as pl
from jax.experimental.pallas import tpu as pltpu

_CHUNK_ROWS = 4096
_NBUF = 4


def _body(i_ref, a_ref, x_hbm, o_hbm, xbuf, obuf, insem, outsem):
    n = x_hbm.shape[0]
    c = _CHUNK_ROWS
    nchunks = n // c

    def get_in(k, slot):
        return pltpu.make_async_copy(
            x_hbm.at[pl.ds(k * c, c), :], xbuf.at[slot], insem.at[slot]
        )

    def put_out(k, slot):
        return pltpu.make_async_copy(
            obuf.at[slot], o_hbm.at[pl.ds(k * c, c), :], outsem.at[slot]
        )

    nbuf = _NBUF
    for k in range(min(nbuf - 1, nchunks)):
        get_in(k, k % nbuf).start()
    for k in range(nchunks):
        slot = k % nbuf
        if k + nbuf - 1 < nchunks:
            get_in(k + nbuf - 1, (k + nbuf - 1) % nbuf).start()
        get_in(k, slot).wait()
        if k >= nbuf:
            put_out(k - nbuf, slot).wait()
        x = xbuf[slot]
        idx = jnp.broadcast_to(i_ref[0:1, :], x.shape)
        z = jnp.take_along_axis(x, idx, axis=1) - a_ref[0:1, :]
        obuf[slot] = jax.nn.sigmoid(z)
        put_out(k, slot).start()
    for k in range(max(0, nchunks - nbuf), nchunks):
        put_out(k, k % nbuf).wait()


@jax.jit
def kernel(x, a, a_index):
    n, d = x.shape
    idx_b = jnp.broadcast_to(a_index[None, :], (8, d))
    a_b = jnp.broadcast_to(a, (8, d))
    return pl.pallas_call(
        _body,
        in_specs=[
            pl.BlockSpec((8, d), lambda: (0, 0)),
            pl.BlockSpec((8, d), lambda: (0, 0)),
            pl.BlockSpec(memory_space=pl.ANY),
        ],
        out_specs=pl.BlockSpec(memory_space=pl.ANY),
        out_shape=jax.ShapeDtypeStruct((n, d), x.dtype),
        scratch_shapes=[
            pltpu.VMEM((_NBUF, _CHUNK_ROWS, d), x.dtype),
            pltpu.VMEM((_NBUF, _CHUNK_ROWS, d), x.dtype),
            pltpu.SemaphoreType.DMA((_NBUF,)),
            pltpu.SemaphoreType.DMA((_NBUF,)),
        ],
    )(idx_b, a_b, x)
